# Initial kernel scaffold; baseline (speedup 1.0000x reference)
import jax
import jax.numpy as jnp
from jax.experimental import pallas as pl


def kernel(inputs, c_prev, s_prev, Wk, bk, Wv, bv, Wb, bb, Wg, bg):
    raise NotImplementedError("write your pallas kernel here")


# speedup vs baseline: 28.7031x; 28.7031x over previous
import jax
import jax.numpy as jnp
from jax.experimental import pallas as pl
from jax.experimental.pallas import tpu as pltpu

_L = 64


def _scan_kernel(x_ref, cp_ref, sp_ref, wk_ref, bk_ref, wv_ref, bv_ref,
                 wbg_ref, bbg_ref, cseq_ref, sseq_ref,
                 kb_scr, kbs_scr, dec_scr, vc_scr, c_scr, s_scr):
    L, D = x_ref.shape

    @pl.when(pl.program_id(0) == 0)
    def _init():
        c_scr[...] = cp_ref[...]
        s_scr[...] = sp_ref[...]

    hp = jax.lax.Precision.HIGHEST
    x = x_ref[...]
    dn = (((1,), (0,)), ((), ()))
    keys = jax.lax.dot_general(x, wk_ref[...], dn,
                               preferred_element_type=jnp.float32,
                               precision=hp) + bk_ref[...]
    kb = keys * jax.lax.rsqrt(jnp.sum(keys * keys, axis=1, keepdims=True))
    vals = jax.lax.dot_general(x, wv_ref[...], dn,
                               preferred_element_type=jnp.float32,
                               precision=hp) + bv_ref[...]
    bg = jax.lax.dot_general(x, wbg_ref[...], dn,
                             preferred_element_type=jnp.float32,
                             precision=hp) + bbg_ref[...]
    sg = jax.nn.sigmoid(bg)
    beta = sg[:, 0:1]
    gam = sg[:, 1:2]
    kb_scr[...] = kb
    kbs_scr[...] = kb * (beta * gam)
    dec_scr[...] = jnp.broadcast_to(1.0 - gam, dec_scr.shape)
    vc_scr[...] = jnp.broadcast_to(vals[:, :, None], vc_scr.shape)

    def step(t, carry):
        kb_row = kb_scr[t, :][None, :]
        kbs_row = kbs_scr[t, :][None, :]
        dec_row = dec_scr[t, :][None, :]
        vcol = vc_scr[t, :, 0:1]
        c = c_scr[...]
        m = jnp.sum(c * kb_row, axis=1, keepdims=True)
        nv = vcol - m
        c_new = c * dec_row + nv * kbs_row
        cseq_ref[t] = c_new
        c_scr[...] = c_new
        s_new = s_scr[...] * dec_row + kbs_row
        sseq_ref[pl.ds(t, 1), :] = s_new
        s_scr[...] = s_new
        return carry

    jax.lax.fori_loop(0, L, step, 0)


def kernel(inputs, c_prev, s_prev, Wk, bk, Wv, bv, Wb, bb, Wg, bg,
           interpret=False):
    T, D = inputs.shape
    wbg = jnp.concatenate([Wb, Wg], axis=1)
    bbg = jnp.concatenate([bb, bg]).reshape(1, 2)
    nc = T // _L

    c_seq, s_seq = pl.pallas_call(
        _scan_kernel,
        grid=(nc,),
        in_specs=[
            pl.BlockSpec((_L, D), lambda i: (i, 0)),
            pl.BlockSpec((D, D), lambda i: (0, 0)),
            pl.BlockSpec((1, D), lambda i: (0, 0)),
            pl.BlockSpec((D, D), lambda i: (0, 0)),
            pl.BlockSpec((1, D), lambda i: (0, 0)),
            pl.BlockSpec((D, D), lambda i: (0, 0)),
            pl.BlockSpec((1, D), lambda i: (0, 0)),
            pl.BlockSpec((D, 2), lambda i: (0, 0)),
            pl.BlockSpec((1, 2), lambda i: (0, 0)),
        ],
        out_specs=[
            pl.BlockSpec((_L, D, D), lambda i: (i, 0, 0)),
            pl.BlockSpec((_L, D), lambda i: (i, 0)),
        ],
        out_shape=[
            jax.ShapeDtypeStruct((T, D, D), jnp.float32),
            jax.ShapeDtypeStruct((T, D), jnp.float32),
        ],
        scratch_shapes=[
            pltpu.VMEM((_L, D), jnp.float32),
            pltpu.VMEM((_L, D), jnp.float32),
            pltpu.VMEM((_L, D), jnp.float32),
            pltpu.VMEM((_L, D, 128), jnp.float32),
            pltpu.VMEM((D, D), jnp.float32),
            pltpu.VMEM((1, D), jnp.float32),
        ],
        compiler_params=pltpu.CompilerParams(
            dimension_semantics=("arbitrary",),
            vmem_limit_bytes=56 * 1024 * 1024,
        ),
        name="memory_layer_scan",
        interpret=interpret,
    )(inputs, c_prev, s_prev.reshape(1, D), Wk, bk.reshape(1, D),
      Wv, bv.reshape(1, D), wbg, bbg)
    return c_seq, s_seq


# speedup vs baseline: 46.1283x; 1.6071x over previous
import jax
import jax.numpy as jnp
from jax.experimental import pallas as pl
from jax.experimental.pallas import tpu as pltpu

_L = 64


def _scan_kernel(x_ref, cp_ref, sp_ref, wk_ref, bk_ref, wv_ref, bv_ref,
                 wbg_ref, bbg_ref, cseq_ref, sseq_ref,
                 kb_scr, kbs_scr, dec_scr, vT_scr, oh_scr, c_scr, s_scr):
    L, D = x_ref.shape

    @pl.when(pl.program_id(0) == 0)
    def _init():
        c_scr[...] = cp_ref[...]
        s_scr[...] = sp_ref[...]

    hp = jax.lax.Precision.HIGHEST
    x = x_ref[...]
    dn = (((1,), (0,)), ((), ()))
    dn_t = (((0,), (1,)), ((), ()))
    keys = jax.lax.dot_general(x, wk_ref[...], dn,
                               preferred_element_type=jnp.float32,
                               precision=hp) + bk_ref[...]
    kb = keys * jax.lax.rsqrt(jnp.sum(keys * keys, axis=1, keepdims=True))
    valsT = jax.lax.dot_general(wv_ref[...], x, dn_t,
                                preferred_element_type=jnp.float32,
                                precision=hp) + bv_ref[...]
    bg = jax.lax.dot_general(x, wbg_ref[...], dn,
                             preferred_element_type=jnp.float32,
                             precision=hp) + bbg_ref[...]
    sg = jax.nn.sigmoid(bg)
    beta = sg[:, 0:1]
    gam = sg[:, 1:2]
    kb_scr[...] = kb
    kbs_scr[...] = kb * (beta * gam)
    dec_scr[...] = jnp.broadcast_to(1.0 - gam, dec_scr.shape)
    vT_scr[...] = valsT
    oh_scr[...] = (jax.lax.broadcasted_iota(jnp.int32, (L, L), 0) ==
                   jax.lax.broadcasted_iota(jnp.int32, (L, L), 1)
                   ).astype(jnp.float32)

    def _step(t, c):
        kb_row = kb_scr[t, :][None, :]
        kbs_row = kbs_scr[t, :][None, :]
        dec_row = dec_scr[t, :][None, :]
        oh_row = oh_scr[t, :][None, :]
        vcol = jnp.sum(vT_scr[...] * oh_row, axis=1, keepdims=True)
        m = jnp.sum(c * kb_row, axis=1, keepdims=True)
        nv = vcol - m
        c_new = c * dec_row + nv * kbs_row
        cseq_ref[t] = c_new
        s_new = s_scr[...] * dec_row + kbs_row
        sseq_ref[pl.ds(t, 1), :] = s_new
        s_scr[...] = s_new

    _step(0, c_scr[...])

    def body(t, carry):
        _step(t, cseq_ref[t - 1])
        return carry

    jax.lax.fori_loop(1, L, body, 0)
    c_scr[...] = cseq_ref[L - 1]


def kernel(inputs, c_prev, s_prev, Wk, bk, Wv, bv, Wb, bb, Wg, bg,
           interpret=False):
    T, D = inputs.shape
    wbg = jnp.concatenate([Wb, Wg], axis=1)
    bbg = jnp.concatenate([bb, bg]).reshape(1, 2)
    nc = T // _L

    c_seq, s_seq = pl.pallas_call(
        _scan_kernel,
        grid=(nc,),
        in_specs=[
            pl.BlockSpec((_L, D), lambda i: (i, 0)),
            pl.BlockSpec((D, D), lambda i: (0, 0)),
            pl.BlockSpec((1, D), lambda i: (0, 0)),
            pl.BlockSpec((D, D), lambda i: (0, 0)),
            pl.BlockSpec((1, D), lambda i: (0, 0)),
            pl.BlockSpec((D, D), lambda i: (0, 0)),
            pl.BlockSpec((D, 1), lambda i: (0, 0)),
            pl.BlockSpec((D, 2), lambda i: (0, 0)),
            pl.BlockSpec((1, 2), lambda i: (0, 0)),
        ],
        out_specs=[
            pl.BlockSpec((_L, D, D), lambda i: (i, 0, 0)),
            pl.BlockSpec((_L, D), lambda i: (i, 0)),
        ],
        out_shape=[
            jax.ShapeDtypeStruct((T, D, D), jnp.float32),
            jax.ShapeDtypeStruct((T, D), jnp.float32),
        ],
        scratch_shapes=[
            pltpu.VMEM((_L, D), jnp.float32),
            pltpu.VMEM((_L, D), jnp.float32),
            pltpu.VMEM((_L, D), jnp.float32),
            pltpu.VMEM((D, _L), jnp.float32),
            pltpu.VMEM((_L, _L), jnp.float32),
            pltpu.VMEM((D, D), jnp.float32),
            pltpu.VMEM((1, D), jnp.float32),
        ],
        compiler_params=pltpu.CompilerParams(
            dimension_semantics=("arbitrary",),
            vmem_limit_bytes=56 * 1024 * 1024,
        ),
        name="memory_layer_scan",
        interpret=interpret,
    )(inputs, c_prev, s_prev.reshape(1, D), Wk, bk.reshape(1, D),
      Wv, bv.reshape(D, 1), wbg, bbg)
    return c_seq, s_seq


# speedup vs baseline: 46.6475x; 1.0113x over previous
import jax
import jax.numpy as jnp
from jax.experimental import pallas as pl
from jax.experimental.pallas import tpu as pltpu

_L = 64


def _scan_kernel(x_ref, cp_ref, sp_ref, wk_ref, bk_ref, wv_ref, bv_ref,
                 wbg_ref, bbg_ref, cseq_ref, sseq_ref,
                 kb_scr, kbs_scr, dec_scr, uT_scr, oh_scr, c_scr, s_scr):
    L, D = x_ref.shape

    @pl.when(pl.program_id(0) == 0)
    def _init():
        c_scr[...] = cp_ref[...]
        s_scr[...] = sp_ref[...]

    hi = jax.lax.Precision.HIGHEST
    x = x_ref[...]
    dn = (((1,), (0,)), ((), ()))
    dn_t = (((0,), (1,)), ((), ()))
    dn_rr = (((1,), (1,)), ((), ()))

    def dot(a, b, d):
        return jax.lax.dot_general(a, b, d, preferred_element_type=jnp.float32,
                                   precision=hi)

    keys = dot(x, wk_ref[...], dn) + bk_ref[...]
    kb = keys * jax.lax.rsqrt(jnp.sum(keys * keys, axis=1, keepdims=True))
    valsT = dot(wv_ref[...], x, dn_t) + bv_ref[...]
    bg = dot(x, wbg_ref[...], dn) + bbg_ref[...]
    bgT = dot(wbg_ref[...], x, dn_t) + bbg_ref[...].reshape(2, 1)

    sg = jax.nn.sigmoid(bg)
    beta_c, gam_c = sg[:, 0:1], sg[:, 1:2]
    sgT = jax.nn.sigmoid(bgT)
    beta_r, gam_r = sgT[0:1, :], sgT[1:2, :]
    dec_c = jnp.maximum(1.0 - gam_c, 1e-30)
    dec_r = jnp.maximum(1.0 - gam_r, 1e-30)
    sc_r = beta_r * gam_r

    ii = jax.lax.broadcasted_iota(jnp.int32, (L, L), 0)
    jj = jax.lax.broadcasted_iota(jnp.int32, (L, L), 1)
    tril = (jj <= ii).astype(jnp.float32)
    triu = (ii <= jj).astype(jnp.float32)
    oh_scr[...] = (ii == jj).astype(jnp.float32)
    lg_c = jnp.log(dec_c)
    lg_r = jnp.log(dec_r)
    s_incl_c = jnp.sum(tril * lg_r, axis=1, keepdims=True)
    s_incl_r = jnp.sum(triu * lg_c, axis=0, keepdims=True)
    s_excl_r = s_incl_r - lg_r
    e1_r = jnp.exp(s_excl_r)

    gram = dot(kb, kb, dn_rr)
    kc0T = dot(c_scr[...], kb, dn_rr)

    arg = jnp.where(ii < jj, s_excl_r - s_incl_c, -1e30)
    wT = jnp.exp(arg) * gram * sc_r
    rhsT = sc_r * (valsT - e1_r * kc0T)

    p = -wT
    uT = rhsT + dot(rhsT, p, dn)
    for _ in range(5):
        p = dot(p, p, dn)
        uT = uT + dot(uT, p, dn)
    uT_scr[...] = uT

    kb_scr[...] = kb
    kbs_scr[...] = kb * (beta_c * gam_c)
    dec_scr[...] = jnp.broadcast_to(dec_c, dec_scr.shape)

    def _step(t, c):
        kb_row = kb_scr[t, :][None, :]
        kbs_row = kbs_scr[t, :][None, :]
        dec_row = dec_scr[t, :][None, :]
        oh_row = oh_scr[t, :][None, :]
        ucol = jnp.sum(uT_scr[...] * oh_row, axis=1, keepdims=True)
        c_new = c * dec_row + ucol * kb_row
        cseq_ref[t] = c_new
        s_new = s_scr[...] * dec_row + kbs_row
        sseq_ref[pl.ds(t, 1), :] = s_new
        s_scr[...] = s_new

    _step(0, c_scr[...])

    def body(t, carry):
        _step(t, cseq_ref[t - 1])
        return carry

    jax.lax.fori_loop(1, L, body, 0)
    c_scr[...] = cseq_ref[L - 1]


def kernel(inputs, c_prev, s_prev, Wk, bk, Wv, bv, Wb, bb, Wg, bg,
           interpret=False):
    T, D = inputs.shape
    wbg = jnp.concatenate([Wb, Wg], axis=1)
    bbg = jnp.concatenate([bb, bg]).reshape(1, 2)
    nc = T // _L

    c_seq, s_seq = pl.pallas_call(
        _scan_kernel,
        grid=(nc,),
        in_specs=[
            pl.BlockSpec((_L, D), lambda i: (i, 0)),
            pl.BlockSpec((D, D), lambda i: (0, 0)),
            pl.BlockSpec((1, D), lambda i: (0, 0)),
            pl.BlockSpec((D, D), lambda i: (0, 0)),
            pl.BlockSpec((1, D), lambda i: (0, 0)),
            pl.BlockSpec((D, D), lambda i: (0, 0)),
            pl.BlockSpec((D, 1), lambda i: (0, 0)),
            pl.BlockSpec((D, 2), lambda i: (0, 0)),
            pl.BlockSpec((1, 2), lambda i: (0, 0)),
        ],
        out_specs=[
            pl.BlockSpec((_L, D, D), lambda i: (i, 0, 0)),
            pl.BlockSpec((_L, D), lambda i: (i, 0)),
        ],
        out_shape=[
            jax.ShapeDtypeStruct((T, D, D), jnp.float32),
            jax.ShapeDtypeStruct((T, D), jnp.float32),
        ],
        scratch_shapes=[
            pltpu.VMEM((_L, D), jnp.float32),
            pltpu.VMEM((_L, D), jnp.float32),
            pltpu.VMEM((_L, D), jnp.float32),
            pltpu.VMEM((D, _L), jnp.float32),
            pltpu.VMEM((_L, _L), jnp.float32),
            pltpu.VMEM((D, D), jnp.float32),
            pltpu.VMEM((1, D), jnp.float32),
        ],
        compiler_params=pltpu.CompilerParams(
            dimension_semantics=("arbitrary",),
            vmem_limit_bytes=56 * 1024 * 1024,
        ),
        name="memory_layer_scan",
        interpret=interpret,
    )(inputs, c_prev, s_prev.reshape(1, D), Wk, bk.reshape(1, D),
      Wv, bv.reshape(D, 1), wbg, bbg)
    return c_seq, s_seq
